# Initial kernel scaffold; baseline (speedup 1.0000x reference)
import jax
import jax.numpy as jnp
from jax import lax
from jax.experimental import pallas as pl
from jax.experimental.pallas import tpu as pltpu
from jax.experimental.pallas import tpu_sc as plsc

N_NODES = 10000
D = 128
L = 16
NC, NS = 2, 16
NW = NC * NS
EPT = 10240
E_PAD = NW * EPT
BLK = 128
OUTER = 8
N_OUTER = EPT // (OUTER * BLK)
ROWS_PT = N_NODES // NS
ZROWS = 125


def _sc_body(x_hbm, col_hbm, row_hbm, deg_hbm, out_hbm,
             acc, zbuf, colv, rowv, degv, gbuf, sem):
    cid = lax.axis_index("c")
    sid = lax.axis_index("s")
    wid = cid * NS + sid

    zeros16 = jnp.zeros((L,), jnp.float32)

    def zrow(r, carry):
        for c in range(D // L):
            zbuf[r, pl.ds(c * L, L)] = zeros16
        return carry

    lax.fori_loop(0, ZROWS, zrow, 0)
    base = sid * ROWS_PT
    for k in range(ROWS_PT // ZROWS):
        pltpu.sync_copy(zbuf, acc.at[pl.ds(base + k * ZROWS, ZROWS)])
    plsc.subcore_barrier()

    def outer(o, carry):
        blk0 = wid * (EPT // BLK) + o * OUTER
        e0 = blk0 * BLK
        pltpu.sync_copy(col_hbm.at[pl.ds(blk0, OUTER)], colv)
        pltpu.sync_copy(row_hbm.at[pl.ds(blk0, OUTER)], rowv)
        pltpu.sync_copy(deg_hbm.at[pl.ds(e0, OUTER * BLK)], degv)
        for b in range(OUTER):
            pltpu.async_copy(x_hbm.at[colv.at[b]], gbuf, sem).wait()

            def srow(j, c2):
                d = degv[b * BLK + j]
                for c in range(D // L):
                    sl = pl.ds(c * L, L)
                    gbuf[j, sl] = gbuf[j, sl] * d
                return c2

            lax.fori_loop(0, BLK, srow, 0)
            pltpu.sync_copy(gbuf, acc.at[rowv.at[b]], add=True)
        return carry

    lax.fori_loop(0, N_OUTER, outer, 0)
    plsc.subcore_barrier()

    pltpu.sync_copy(acc.at[pl.ds(base, ROWS_PT)],
                    out_hbm.at[cid, pl.ds(base, ROWS_PT)])


_sc_agg = pl.kernel(
    _sc_body,
    out_type=jax.ShapeDtypeStruct((NC, N_NODES, D), jnp.float32),
    mesh=plsc.VectorSubcoreMesh(core_axis_name="c", subcore_axis_name="s"),
    scratch_types=[
        pltpu.VMEM_SHARED((N_NODES, D), jnp.float32),
        pltpu.VMEM((ZROWS, D), jnp.float32),
        pltpu.VMEM((OUTER, BLK), jnp.int32),
        pltpu.VMEM((OUTER, BLK), jnp.int32),
        pltpu.VMEM((OUTER * BLK,), jnp.float32),
        pltpu.VMEM((BLK, D), jnp.float32),
        pltpu.SemaphoreType.DMA,
    ],
)


def _mm_body(p_ref, w_ref, o_ref):
    p = p_ref[0] + p_ref[1]
    o_ref[...] = jnp.dot(p, w_ref[...], preferred_element_type=jnp.float32)


def _matmul(partials, W):
    BM = 2000
    return pl.pallas_call(
        _mm_body,
        grid=(N_NODES // BM,),
        in_specs=[pl.BlockSpec((NC, BM, D), lambda i: (0, i, 0)),
                  pl.BlockSpec((D, D), lambda i: (0, 0))],
        out_specs=pl.BlockSpec((BM, D), lambda i: (i, 0)),
        out_shape=jax.ShapeDtypeStruct((N_NODES, D), jnp.float32),
    )(partials, W)


@jax.jit
def _impl(X, row_index, column_index, degrees, W):
    col = column_index.astype(jnp.int32)
    row = row_index.astype(jnp.int32)
    deg = degrees.astype(jnp.float32)
    pad = E_PAD - col.shape[0]
    col = jnp.concatenate([col, jnp.zeros((pad,), jnp.int32)])
    row = jnp.concatenate([row, jnp.zeros((pad,), jnp.int32)])
    deg = jnp.concatenate([deg, jnp.zeros((pad,), jnp.float32)])
    col = col.reshape(E_PAD // BLK, BLK)
    row = row.reshape(E_PAD // BLK, BLK)
    partials = _sc_agg(X, col, row, deg)
    return _matmul(partials, W)


def kernel(X, row_index, column_index, degrees, W):
    return _impl(X, row_index, column_index, degrees, W)


# speedup vs baseline: 3.6882x; 3.6882x over previous
import jax
import jax.numpy as jnp
from jax import lax
from jax.experimental import pallas as pl
from jax.experimental.pallas import tpu as pltpu
from jax.experimental.pallas import tpu_sc as plsc

N_NODES = 10000
D = 128
L = 16
NC, NS = 2, 16
NW = NC * NS
EPT = 10240
E_PAD = NW * EPT
BLK = 128
OUTER = 8
N_OUTER = EPT // (OUTER * BLK)
ROWS_PT = 624
REM_BASE = NS * ROWS_PT
REM = N_NODES - REM_BASE
ZROWS = 208


def _sc_body(x_hbm, col_hbm, row_hbm, deg_hbm, out_hbm,
             acc, zbuf, colv, rowv, degv, gbuf, sem):
    cid = lax.axis_index("c")
    sid = lax.axis_index("s")
    wid = cid * NS + sid

    zeros16 = jnp.zeros((L,), jnp.float32)

    def zrow(r, carry):
        for c in range(D // L):
            zbuf[r, pl.ds(c * L, L)] = zeros16
        return carry

    lax.fori_loop(0, ZROWS, zrow, 0)
    base = sid * ROWS_PT
    for k in range(ROWS_PT // ZROWS):
        pltpu.sync_copy(zbuf, acc.at[pl.ds(base + k * ZROWS, ZROWS)])

    @pl.when(sid == 0)
    def _():
        pltpu.sync_copy(zbuf.at[pl.ds(0, REM)], acc.at[pl.ds(REM_BASE, REM)])

    plsc.subcore_barrier()

    def outer(o, carry):
        blk0 = wid * (EPT // BLK) + o * OUTER
        e0 = blk0 * BLK
        pltpu.sync_copy(col_hbm.at[pl.ds(blk0, OUTER)], colv)
        pltpu.sync_copy(row_hbm.at[pl.ds(blk0, OUTER)], rowv)
        pltpu.sync_copy(deg_hbm.at[pl.ds(e0, OUTER * BLK)], degv)
        for b in range(OUTER):
            pltpu.async_copy(x_hbm.at[colv.at[b]], gbuf, sem).wait()

            def sgrp(g, c2):
                dv = degv[pl.ds(b * BLK + g * L, L)]
                for k in range(L):
                    d = dv[k]
                    j = g * L + k
                    for c in range(D // L):
                        sl = pl.ds(c * L, L)
                        gbuf[j, sl] = gbuf[j, sl] * d
                return c2

            lax.fori_loop(0, BLK // L, sgrp, 0)
            pltpu.sync_copy(gbuf, acc.at[rowv.at[b]], add=True)
        return carry

    lax.fori_loop(0, N_OUTER, outer, 0)
    plsc.subcore_barrier()

    pltpu.sync_copy(acc.at[pl.ds(base, ROWS_PT)],
                    out_hbm.at[cid, pl.ds(base, ROWS_PT)])

    @pl.when(sid == 0)
    def _():
        pltpu.sync_copy(acc.at[pl.ds(REM_BASE, REM)],
                        out_hbm.at[cid, pl.ds(REM_BASE, REM)])


_sc_agg = pl.kernel(
    _sc_body,
    out_type=jax.ShapeDtypeStruct((NC, N_NODES, D), jnp.float32),
    mesh=plsc.VectorSubcoreMesh(core_axis_name="c", subcore_axis_name="s"),
    scratch_types=[
        pltpu.VMEM_SHARED((N_NODES, D), jnp.float32),
        pltpu.VMEM((ZROWS, D), jnp.float32),
        pltpu.VMEM((OUTER, BLK), jnp.int32),
        pltpu.VMEM((OUTER, BLK), jnp.int32),
        pltpu.VMEM((OUTER * BLK,), jnp.float32),
        pltpu.VMEM((BLK, D), jnp.float32),
        pltpu.SemaphoreType.DMA,
    ],
)


def _mm_body(p_ref, w_ref, o_ref):
    p = p_ref[0] + p_ref[1]
    o_ref[...] = jnp.dot(p, w_ref[...], preferred_element_type=jnp.float32)


def _matmul(partials, W):
    BM = 2000
    return pl.pallas_call(
        _mm_body,
        grid=(N_NODES // BM,),
        in_specs=[pl.BlockSpec((NC, BM, D), lambda i: (0, i, 0)),
                  pl.BlockSpec((D, D), lambda i: (0, 0))],
        out_specs=pl.BlockSpec((BM, D), lambda i: (i, 0)),
        out_shape=jax.ShapeDtypeStruct((N_NODES, D), jnp.float32),
    )(partials, W)


@jax.jit
def _impl(X, row_index, column_index, degrees, W):
    col = column_index.astype(jnp.int32)
    row = row_index.astype(jnp.int32)
    deg = degrees.astype(jnp.float32)
    pad = E_PAD - col.shape[0]
    col = jnp.concatenate([col, jnp.zeros((pad,), jnp.int32)])
    row = jnp.concatenate([row, jnp.zeros((pad,), jnp.int32)])
    deg = jnp.concatenate([deg, jnp.zeros((pad,), jnp.float32)])
    col = col.reshape(E_PAD // BLK, BLK)
    row = row.reshape(E_PAD // BLK, BLK)
    partials = _sc_agg(X, col, row, deg)
    return _matmul(partials, W)


def kernel(X, row_index, column_index, degrees, W):
    return _impl(X, row_index, column_index, degrees, W)


# speedup vs baseline: 4.3722x; 1.1855x over previous
import jax
import jax.numpy as jnp
from jax import lax
from jax.experimental import pallas as pl
from jax.experimental.pallas import tpu as pltpu
from jax.experimental.pallas import tpu_sc as plsc

N_NODES = 10000
D = 128
L = 16
NC, NS = 2, 16
NW = NC * NS
EPT = 10240
E_PAD = NW * EPT
BLK = 128
OUTER = 8
N_OUTER = EPT // (OUTER * BLK)
ROWS_PT = 624
REM_BASE = NS * ROWS_PT
REM = N_NODES - REM_BASE
ZROWS = 24
NBLK = EPT // BLK
SBLK = 16
N_STAGE = NBLK // SBLK


def _sc_body(x_hbm, col_hbm, row_hbm, deg_hbm, out_hbm,
             acc, zbuf, colv, rowv, degv, gbuf0, gbuf1,
             sem_g0, sem_g1, sem_s0, sem_s1):
    cid = lax.axis_index("c")
    sid = lax.axis_index("s")
    wid = cid * NS + sid

    zeros16 = jnp.zeros((L,), jnp.float32)

    def zrow(r, carry):
        for c in range(D // L):
            zbuf[r, pl.ds(c * L, L)] = zeros16
        return carry

    lax.fori_loop(0, ZROWS, zrow, 0)
    base = sid * ROWS_PT
    for k in range(ROWS_PT // ZROWS):
        pltpu.sync_copy(zbuf, acc.at[pl.ds(base + k * ZROWS, ZROWS)])

    @pl.when(sid == 0)
    def _():
        pltpu.sync_copy(zbuf.at[pl.ds(0, REM)], acc.at[pl.ds(REM_BASE, REM)])

    plsc.subcore_barrier()

    gbufs = (gbuf0, gbuf1)
    gsems = (sem_g0, sem_g1)
    ssems = (sem_s0, sem_s1)

    def gather(b, p):
        pltpu.async_copy(x_hbm.at[colv.at[b]], gbufs[p], gsems[p])

    def wait_gather(p):
        pltpu.make_async_copy(x_hbm.at[colv.at[0]], gbufs[p], gsems[p]).wait()

    def scatter(b, p):
        pltpu.async_copy(gbufs[p], acc.at[rowv.at[b]], ssems[p], add=True)

    def wait_scatter(p):
        pltpu.make_async_copy(gbufs[p], acc.at[rowv.at[0]], ssems[p]).wait()

    def scale(p, b):
        buf = gbufs[p]

        def sgrp(g, c2):
            dv = degv[pl.ds(b * BLK + g * L, L)]
            for k in range(L):
                d = dv[k]
                j = g * L + k
                for c in range(D // L):
                    sl = pl.ds(c * L, L)
                    buf[j, sl] = buf[j, sl] * d
            return c2

        lax.fori_loop(0, BLK // L, sgrp, 0)

    def stage(s, carry):
        blk0 = wid * NBLK + s * SBLK
        pltpu.sync_copy(col_hbm.at[pl.ds(blk0, SBLK)], colv)
        pltpu.sync_copy(row_hbm.at[pl.ds(blk0, SBLK)], rowv)
        pltpu.sync_copy(deg_hbm.at[pl.ds(blk0 * BLK, SBLK * BLK)], degv)
        gather(0, 0)
        gather(1, 1)

        def pipe(i, c2):
            b0 = 2 * i
            b1 = 2 * i + 1
            wait_gather(0)
            scale(0, b0)
            scatter(b0, 0)
            wait_gather(1)
            scale(1, b1)
            scatter(b1, 1)

            @pl.when(i < SBLK // 2 - 1)
            def _():
                wait_scatter(0)
                gather(b0 + 2, 0)
                wait_scatter(1)
                gather(b1 + 2, 1)

            return c2

        lax.fori_loop(0, SBLK // 2, pipe, 0)
        wait_scatter(0)
        wait_scatter(1)
        return carry

    lax.fori_loop(0, N_STAGE, stage, 0)
    plsc.subcore_barrier()

    pltpu.sync_copy(acc.at[pl.ds(base, ROWS_PT)],
                    out_hbm.at[cid, pl.ds(base, ROWS_PT)])

    @pl.when(sid == 0)
    def _():
        pltpu.sync_copy(acc.at[pl.ds(REM_BASE, REM)],
                        out_hbm.at[cid, pl.ds(REM_BASE, REM)])


_sc_agg = pl.kernel(
    _sc_body,
    out_type=jax.ShapeDtypeStruct((NC, N_NODES, D), jnp.float32),
    mesh=plsc.VectorSubcoreMesh(core_axis_name="c", subcore_axis_name="s"),
    scratch_types=[
        pltpu.VMEM_SHARED((N_NODES, D), jnp.float32),
        pltpu.VMEM((ZROWS, D), jnp.float32),
        pltpu.VMEM((SBLK, BLK), jnp.int32),
        pltpu.VMEM((SBLK, BLK), jnp.int32),
        pltpu.VMEM((SBLK * BLK,), jnp.float32),
        pltpu.VMEM((BLK, D), jnp.float32),
        pltpu.VMEM((BLK, D), jnp.float32),
        pltpu.SemaphoreType.DMA,
        pltpu.SemaphoreType.DMA,
        pltpu.SemaphoreType.DMA,
        pltpu.SemaphoreType.DMA,
    ],
)


def _mm_body(p_ref, w_ref, o_ref):
    p = p_ref[0] + p_ref[1]
    o_ref[...] = jnp.dot(p, w_ref[...], preferred_element_type=jnp.float32)


def _matmul(partials, W):
    BM = 2000
    return pl.pallas_call(
        _mm_body,
        grid=(N_NODES // BM,),
        in_specs=[pl.BlockSpec((NC, BM, D), lambda i: (0, i, 0)),
                  pl.BlockSpec((D, D), lambda i: (0, 0))],
        out_specs=pl.BlockSpec((BM, D), lambda i: (i, 0)),
        out_shape=jax.ShapeDtypeStruct((N_NODES, D), jnp.float32),
    )(partials, W)


@jax.jit
def _impl(X, row_index, column_index, degrees, W):
    col = column_index.astype(jnp.int32)
    row = row_index.astype(jnp.int32)
    deg = degrees.astype(jnp.float32)
    pad = E_PAD - col.shape[0]
    col = jnp.concatenate([col, jnp.zeros((pad,), jnp.int32)])
    row = jnp.concatenate([row, jnp.zeros((pad,), jnp.int32)])
    deg = jnp.concatenate([deg, jnp.zeros((pad,), jnp.float32)])
    col = col.reshape(E_PAD // BLK, BLK)
    row = row.reshape(E_PAD // BLK, BLK)
    partials = _sc_agg(X, col, row, deg)
    return _matmul(partials, W)


def kernel(X, row_index, column_index, degrees, W):
    return _impl(X, row_index, column_index, degrees, W)


# speedup vs baseline: 10.4264x; 2.3847x over previous
import jax
import jax.numpy as jnp
from jax import lax
from jax.experimental import pallas as pl
from jax.experimental.pallas import tpu as pltpu
from jax.experimental.pallas import tpu_sc as plsc

N_NODES = 10000
D = 128
L = 16
NC, NS = 2, 16
NW = NC * NS
EPT = 10240
E_PAD = NW * EPT
BLK = 128
OUTER = 8
N_OUTER = EPT // (OUTER * BLK)
ROWS_PT = 624
REM_BASE = NS * ROWS_PT
REM = N_NODES - REM_BASE
ZROWS = 24
NBLK = EPT // BLK
SBLK = 16
N_STAGE = NBLK // SBLK


def _sc_body(x_hbm, col_hbm, row_hbm, deg_hbm, out_hbm,
             acc, zbuf, colv, rowv, degv, gbuf0, gbuf1,
             sem_g0, sem_g1, sem_s0, sem_s1):
    cid = lax.axis_index("c")
    sid = lax.axis_index("s")
    wid = cid * NS + sid

    zeros16 = jnp.zeros((L,), jnp.float32)

    def zrow(r, carry):
        for c in range(D // L):
            zbuf[r, pl.ds(c * L, L)] = zeros16
        return carry

    lax.fori_loop(0, ZROWS, zrow, 0)
    base = sid * ROWS_PT
    for k in range(ROWS_PT // ZROWS):
        pltpu.sync_copy(zbuf, acc.at[pl.ds(base + k * ZROWS, ZROWS)])

    @pl.when(sid == 0)
    def _():
        pltpu.sync_copy(zbuf.at[pl.ds(0, REM)], acc.at[pl.ds(REM_BASE, REM)])

    plsc.subcore_barrier()

    gbufs = (gbuf0, gbuf1)
    gsems = (sem_g0, sem_g1)
    ssems = (sem_s0, sem_s1)

    def gather(b, p):
        pltpu.async_copy(x_hbm.at[colv.at[b]], gbufs[p], gsems[p])

    def wait_gather(p):
        pltpu.make_async_copy(x_hbm.at[colv.at[0]], gbufs[p], gsems[p]).wait()

    def scatter(b, p):
        pltpu.async_copy(gbufs[p], acc.at[rowv.at[b]], ssems[p], add=True)

    def wait_scatter(p):
        pltpu.make_async_copy(gbufs[p], acc.at[rowv.at[0]], ssems[p]).wait()

    def scale(p, b):
        buf = gbufs[p]

        def sgrp(g, c2):
            dv = degv[pl.ds(b * BLK + g * L, L)]
            for k in range(L):
                d = dv[k]
                j = g * L + k
                for c in range(D // L):
                    sl = pl.ds(c * L, L)
                    buf[j, sl] = buf[j, sl] * d
            return c2

        lax.fori_loop(0, BLK // L, sgrp, 0)

    def stage(s, carry):
        blk0 = wid * NBLK + s * SBLK
        pltpu.sync_copy(col_hbm.at[pl.ds(blk0, SBLK)], colv)
        pltpu.sync_copy(row_hbm.at[pl.ds(blk0, SBLK)], rowv)
        pltpu.sync_copy(deg_hbm.at[pl.ds(blk0 * BLK, SBLK * BLK)], degv)
        gather(0, 0)
        gather(1, 1)

        def pipe(i, c2):
            b0 = 2 * i
            b1 = 2 * i + 1
            wait_gather(0)
            scale(0, b0)
            scatter(b0, 0)
            wait_gather(1)
            scale(1, b1)
            scatter(b1, 1)

            @pl.when(i < SBLK // 2 - 1)
            def _():
                wait_scatter(0)
                gather(b0 + 2, 0)
                wait_scatter(1)
                gather(b1 + 2, 1)

            return c2

        lax.fori_loop(0, SBLK // 2, pipe, 0)
        wait_scatter(0)
        wait_scatter(1)
        return carry

    lax.fori_loop(0, N_STAGE, stage, 0)
    plsc.subcore_barrier()

    pltpu.sync_copy(acc.at[pl.ds(base, ROWS_PT)],
                    out_hbm.at[cid, pl.ds(base, ROWS_PT)])

    @pl.when(sid == 0)
    def _():
        pltpu.sync_copy(acc.at[pl.ds(REM_BASE, REM)],
                        out_hbm.at[cid, pl.ds(REM_BASE, REM)])


_sc_agg = pl.kernel(
    _sc_body,
    out_type=jax.ShapeDtypeStruct((NC, N_NODES, D), jnp.float32),
    mesh=plsc.VectorSubcoreMesh(core_axis_name="c", subcore_axis_name="s"),
    scratch_types=[
        pltpu.VMEM_SHARED((N_NODES, D), jnp.float32),
        pltpu.VMEM((ZROWS, D), jnp.float32),
        pltpu.VMEM((SBLK, BLK), jnp.int32),
        pltpu.VMEM((SBLK, BLK), jnp.int32),
        pltpu.VMEM((SBLK * BLK,), jnp.float32),
        pltpu.VMEM((BLK, D), jnp.float32),
        pltpu.VMEM((BLK, D), jnp.float32),
        pltpu.SemaphoreType.DMA,
        pltpu.SemaphoreType.DMA,
        pltpu.SemaphoreType.DMA,
        pltpu.SemaphoreType.DMA,
    ],
)


def _mm_body(p_ref, w_ref, o_ref):
    p = p_ref[0] + p_ref[1]
    o_ref[...] = jnp.dot(p, w_ref[...], preferred_element_type=jnp.float32)


def _matmul(partials, W):
    BM = 2000
    return pl.pallas_call(
        _mm_body,
        grid=(N_NODES // BM,),
        in_specs=[pl.BlockSpec((NC, BM, D), lambda i: (0, i, 0)),
                  pl.BlockSpec((D, D), lambda i: (0, 0))],
        out_specs=pl.BlockSpec((BM, D), lambda i: (i, 0)),
        out_shape=jax.ShapeDtypeStruct((N_NODES, D), jnp.float32),
    )(partials, W)


@jax.jit
def _impl(X, row_index, column_index, degrees, W):
    col = column_index.astype(jnp.int32)
    row = row_index.astype(jnp.int32)
    deg = degrees.astype(jnp.float32)
    pad = E_PAD - col.shape[0]
    spread = jnp.arange(pad, dtype=jnp.int32) % N_NODES
    col = jnp.concatenate([col, spread])
    row = jnp.concatenate([row, spread])
    deg = jnp.concatenate([deg, jnp.zeros((pad,), jnp.float32)])
    col = col.reshape(E_PAD // BLK, BLK)
    row = row.reshape(E_PAD // BLK, BLK)
    partials = _sc_agg(X, col, row, deg)
    return _matmul(partials, W)


def kernel(X, row_index, column_index, degrees, W):
    return _impl(X, row_index, column_index, degrees, W)
